# baseline (speedup 1.0000x reference)
import jax
import jax.numpy as jnp
import numpy as np
from jax import lax
from jax.experimental import pallas as pl
from jax.experimental.pallas import tpu as pltpu
from jax.experimental.pallas import tpu_sc as plsc

_H = _W = 11
_HW = _H * _W
_C = 128
_E = _HW * _C
_B = 4096


def _spiral_perm() -> np.ndarray:
    cen = _H // 2
    pos = [(cen, cen)]
    for r in range(1, cen + 1):
        pos += [(cen - r, w) for w in range(cen - r + 1, cen + r + 1)]
        pos += [(h, cen + r) for h in range(cen - r + 1, cen + r + 1)]
        pos += [(cen + r, w) for w in range(cen - r, cen + r)]
        pos += [(h, cen - r) for h in range(cen - r, cen + r)]
    return np.array([h * _W + w for h, w in pos], dtype=np.int32)


_PERM_PAD = np.zeros((128,), dtype=np.int32)
_PERM_PAD[:_HW] = _spiral_perm()

_NC, _NS = 2, 16
_NW = _NC * _NS
_NB = _B // _NW


_PERM_LIST = [int(v) for v in _spiral_perm()]


def _body(x_hbm, out_hbm, in0, in1, out0, out1, si0, si1, so0, so1):
    wid = lax.axis_index("s") * _NC + lax.axis_index("c")
    base = wid * _NB
    ins, sis = (in0, in1), (si0, si1)
    outs, sos = (out0, out1), (so0, so1)

    iotav = lax.iota(jnp.int32, 16) * _HW

    def xsl(i):
        return x_hbm.at[pl.ds((base + i) * _E, _E)]

    def osl(i):
        return out_hbm.at[pl.ds((base + i) * _E, _E)]

    for s in range(2):
        pltpu.async_copy(xsl(s), ins[s], sis[s])

    def shuffle(inref, outref):
        for k in range(_HW):
            pk = _PERM_LIST[k]
            for ct in range(8):
                vals = plsc.load_gather(inref, [iotav + (ct * 16 * _HW + pk)])
                outref[pl.ds(k * _C + ct * 16, 16)] = vals

    def gloop(g, carry):
        for s in range(2):
            i = 2 * g + s
            pltpu.make_async_copy(xsl(i), ins[s], sis[s]).wait()

            @pl.when(g > 0)
            def _wait_out():
                pltpu.make_async_copy(outs[s], osl(i - 2), sos[s]).wait()

            shuffle(ins[s], outs[s])
            pltpu.async_copy(outs[s], osl(i), sos[s])

            @pl.when(i + 2 < _NB)
            def _next_in():
                pltpu.async_copy(xsl(i + 2), ins[s], sis[s])
        return carry

    lax.fori_loop(0, _NB // 2, gloop, 0)

    g_last = (_NB // 2) - 1
    for s in range(2):
        pltpu.make_async_copy(outs[s], osl(2 * g_last + s), sos[s]).wait()


@jax.jit
def kernel(x):
    xr = x.reshape(_B * _E)
    mesh = plsc.VectorSubcoreMesh(core_axis_name="c", subcore_axis_name="s",
                                  num_cores=_NC)
    out = pl.kernel(
        _body,
        mesh=mesh,
        compiler_params=pltpu.CompilerParams(needs_layout_passes=False),
        out_type=jax.ShapeDtypeStruct((_B * _E,), jnp.float32),
        scratch_types=(
            [pltpu.VMEM((_E,), jnp.float32)] * 4
            + [pltpu.SemaphoreType.DMA] * 4
        ),
    )(xr)
    return out.reshape(_B, _HW, _C)


# speedup vs baseline: 2.4925x; 2.4925x over previous
import jax
import jax.numpy as jnp
import numpy as np
from jax import lax
from jax.experimental import pallas as pl
from jax.experimental.pallas import tpu as pltpu
from jax.experimental.pallas import tpu_sc as plsc

_H = _W = 11
_HW = _H * _W
_C = 128
_E = _HW * _C
_B = 4096


def _spiral_perm() -> np.ndarray:
    cen = _H // 2
    pos = [(cen, cen)]
    for r in range(1, cen + 1):
        pos += [(cen - r, w) for w in range(cen - r + 1, cen + r + 1)]
        pos += [(h, cen + r) for h in range(cen - r + 1, cen + r + 1)]
        pos += [(cen + r, w) for w in range(cen - r, cen + r)]
        pos += [(h, cen - r) for h in range(cen - r, cen + r)]
    return np.array([h * _W + w for h, w in pos], dtype=np.int32)


_PERM_PAD = np.zeros((128,), dtype=np.int32)
_PERM_PAD[:_HW] = _spiral_perm()

_NC, _NS = 2, 16
_NW = _NC * _NS
_NB = _B // _NW


_PERM_LIST = [int(v) for v in _spiral_perm()]


def _body(x_hbm, out_hbm, in0, in1, out0, out1, si0, si1, so0, so1):
    wid = lax.axis_index("s") * _NC + lax.axis_index("c")
    base = wid * _NB
    ins, sis = (in0, in1), (si0, si1)
    outs, sos = (out0, out1), (so0, so1)

    iotav = lax.iota(jnp.int32, 16) * _HW

    def xsl(i):
        return x_hbm.at[base + i]

    def osl(i):
        return out_hbm.at[base + i]

    for s in range(2):
        pltpu.async_copy(xsl(s), ins[s], sis[s])

    def shuffle(inref, outref):
        for k in range(_HW):
            pk = _PERM_LIST[k]
            for ct in range(8):
                vals = plsc.load_gather(inref, [iotav + (ct * 16 * _HW + pk)])
                outref[pl.ds(k * _C + ct * 16, 16)] = vals

    def gloop(g, carry):
        for s in range(2):
            i = 2 * g + s
            pltpu.make_async_copy(xsl(i), ins[s], sis[s]).wait()

            @pl.when(g > 0)
            def _wait_out():
                pltpu.make_async_copy(outs[s], osl(i - 2), sos[s]).wait()

            shuffle(ins[s], outs[s])
            pltpu.async_copy(outs[s], osl(i), sos[s])

            @pl.when(i + 2 < _NB)
            def _next_in():
                pltpu.async_copy(xsl(i + 2), ins[s], sis[s])
        return carry

    lax.fori_loop(0, _NB // 2, gloop, 0)

    g_last = (_NB // 2) - 1
    for s in range(2):
        pltpu.make_async_copy(outs[s], osl(2 * g_last + s), sos[s]).wait()


@jax.jit
def kernel(x):
    xr = x.reshape(_B, _E)
    mesh = plsc.VectorSubcoreMesh(core_axis_name="c", subcore_axis_name="s",
                                  num_cores=_NC)
    out = pl.kernel(
        _body,
        mesh=mesh,
        compiler_params=pltpu.CompilerParams(needs_layout_passes=False,
                                             use_tc_tiling_on_sc=True),
        out_type=jax.ShapeDtypeStruct((_B, _E), jnp.float32),
        scratch_types=(
            [pltpu.VMEM((_E,), jnp.float32)] * 4
            + [pltpu.SemaphoreType.DMA] * 4
        ),
    )(xr)
    return out.reshape(_B, _HW, _C)


# speedup vs baseline: 7.8100x; 3.1334x over previous
import jax
import jax.numpy as jnp
import numpy as np
from jax.experimental import pallas as pl
from jax.experimental.pallas import tpu as pltpu

_H = _W = 11
_HW = _H * _W
_C = 128
_B = 4096
_CB = 64
_NCHUNK = _B // _CB
_DEPTH = 4


def _spiral_perm() -> np.ndarray:
    cen = _H // 2
    pos = [(cen, cen)]
    for r in range(1, cen + 1):
        pos += [(cen - r, w) for w in range(cen - r + 1, cen + r + 1)]
        pos += [(h, cen + r) for h in range(cen - r + 1, cen + r + 1)]
        pos += [(cen + r, w) for w in range(cen - r, cen + r)]
        pos += [(h, cen - r) for h in range(cen - r, cen + r)]
    return np.array([h * _W + w for h, w in pos], dtype=np.int64)


_P = np.zeros((_HW, _HW), dtype=np.float32)
_P[np.arange(_HW), _spiral_perm()] = 1.0


def _body(p_ref, x_hbm, o_hbm, ibufs, obufs, isems, osems):
    def in_copy(i):
        s = i % _DEPTH
        return pltpu.make_async_copy(
            x_hbm.at[pl.ds(i * _CB, _CB)], ibufs.at[s], isems.at[s])

    def out_copy(i):
        s = i % _DEPTH
        return pltpu.make_async_copy(
            obufs.at[s], o_hbm.at[pl.ds(i * _CB, _CB)], osems.at[s])

    for i in range(_DEPTH):
        in_copy(i).start()

    for i in range(_NCHUNK):
        s = i % _DEPTH
        in_copy(i).wait()
        if i >= _DEPTH:
            out_copy(i - _DEPTH).wait()
        xb = ibufs[s]
        xm = xb.reshape(_CB * _C, _HW)
        ym = jax.lax.dot_general(
            xm, p_ref[...], (((1,), (1,)), ((), ())),
            preferred_element_type=jnp.float32)
        obufs[s] = jnp.transpose(ym.reshape(_CB, _C, _HW), (0, 2, 1))
        out_copy(i).start()
        if i + _DEPTH < _NCHUNK:
            in_copy(i + _DEPTH).start()

    for i in range(_NCHUNK - _DEPTH, _NCHUNK):
        out_copy(i).wait()


@jax.jit
def kernel(x):
    xr = x.reshape(_B, _C, _HW)
    return pl.pallas_call(
        _body,
        in_specs=[
            pl.BlockSpec(memory_space=pltpu.VMEM),
            pl.BlockSpec(memory_space=pltpu.HBM),
        ],
        out_specs=pl.BlockSpec(memory_space=pltpu.HBM),
        out_shape=jax.ShapeDtypeStruct((_B, _HW, _C), x.dtype),
        scratch_shapes=[
            pltpu.VMEM((_DEPTH, _CB, _C, _HW), jnp.float32),
            pltpu.VMEM((_DEPTH, _CB, _HW, _C), jnp.float32),
            pltpu.SemaphoreType.DMA((_DEPTH,)),
            pltpu.SemaphoreType.DMA((_DEPTH,)),
        ],
    )(jnp.asarray(_P), xr)


# speedup vs baseline: 7.8197x; 1.0012x over previous
import jax
import jax.numpy as jnp
import numpy as np
from jax.experimental import pallas as pl
from jax.experimental.pallas import tpu as pltpu

_H = _W = 11
_HW = _H * _W
_C = 128
_B = 4096
_CB = 32
_NCHUNK = _B // _CB
_DEPTH = 8


def _spiral_perm() -> np.ndarray:
    cen = _H // 2
    pos = [(cen, cen)]
    for r in range(1, cen + 1):
        pos += [(cen - r, w) for w in range(cen - r + 1, cen + r + 1)]
        pos += [(h, cen + r) for h in range(cen - r + 1, cen + r + 1)]
        pos += [(cen + r, w) for w in range(cen - r, cen + r)]
        pos += [(h, cen - r) for h in range(cen - r, cen + r)]
    return np.array([h * _W + w for h, w in pos], dtype=np.int64)


_P = np.zeros((_HW, _HW), dtype=np.float32)
_P[np.arange(_HW), _spiral_perm()] = 1.0


def _body(p_ref, x_hbm, o_hbm, ibufs, obufs, isems, osems):
    def in_copy(i):
        s = i % _DEPTH
        return pltpu.make_async_copy(
            x_hbm.at[pl.ds(i * _CB, _CB)], ibufs.at[s], isems.at[s])

    def out_copy(i):
        s = i % _DEPTH
        return pltpu.make_async_copy(
            obufs.at[s], o_hbm.at[pl.ds(i * _CB, _CB)], osems.at[s])

    for i in range(_DEPTH):
        in_copy(i).start()

    for i in range(_NCHUNK):
        s = i % _DEPTH
        in_copy(i).wait()
        if i >= _DEPTH:
            out_copy(i - _DEPTH).wait()
        xb = ibufs[s]
        xm = xb.reshape(_CB * _C, _HW)
        ym = jax.lax.dot_general(
            xm, p_ref[...], (((1,), (1,)), ((), ())),
            preferred_element_type=jnp.float32)
        obufs[s] = jnp.transpose(ym.reshape(_CB, _C, _HW), (0, 2, 1))
        out_copy(i).start()
        if i + _DEPTH < _NCHUNK:
            in_copy(i + _DEPTH).start()

    for i in range(_NCHUNK - _DEPTH, _NCHUNK):
        out_copy(i).wait()


@jax.jit
def kernel(x):
    xr = x.reshape(_B, _C, _HW)
    return pl.pallas_call(
        _body,
        in_specs=[
            pl.BlockSpec(memory_space=pltpu.VMEM),
            pl.BlockSpec(memory_space=pltpu.HBM),
        ],
        out_specs=pl.BlockSpec(memory_space=pltpu.HBM),
        out_shape=jax.ShapeDtypeStruct((_B, _HW, _C), x.dtype),
        scratch_shapes=[
            pltpu.VMEM((_DEPTH, _CB, _C, _HW), jnp.float32),
            pltpu.VMEM((_DEPTH, _CB, _HW, _C), jnp.float32),
            pltpu.SemaphoreType.DMA((_DEPTH,)),
            pltpu.SemaphoreType.DMA((_DEPTH,)),
        ],
    )(jnp.asarray(_P), xr)
